# Initial kernel scaffold; baseline (speedup 1.0000x reference)
import jax
import jax.numpy as jnp
from jax.experimental import pallas as pl


def kernel(x, edge_list, edge_attr, W1, b1, Wfc, bfc):
    raise NotImplementedError("write your pallas kernel here")


# speedup vs baseline: 66.1762x; 66.1762x over previous
import functools

import jax
import jax.numpy as jnp
from jax import lax
from jax.experimental import pallas as pl
from jax.experimental.pallas import tpu as pltpu
from jax.experimental.pallas import tpu_sc as plsc

_N = 10000
_E = 320000
_D = 128
_NP = 10240
_NTILES = 16
_NCORES = 2
_NW = _NCORES * _NTILES
_SL = _NP // _NTILES
_CH = 2000
_GRP = _CH // 16
_EA = _E // _NTILES
_EB = _E // _NW


def _mm_body(x_ref, w_ref, o_ref):
    o_ref[...] = jnp.dot(x_ref[...], w_ref[...],
                         preferred_element_type=jnp.float32)


def _fin_body(acc_ref, w_ref, b1_ref, bfc_ref, o_ref):
    p = jnp.sum(acc_ref[...], axis=1)
    r = jnp.maximum(p + b1_ref[...], 0.0)
    logit = jnp.sum(r * w_ref[...]) + bfc_ref[0, 0]
    o_ref[...] = jnp.reshape(jax.nn.sigmoid(logit), (1, 1))


def _sc_body(src_hbm, dst_hbm, ew_hbm, h01_hbm, acc_hbm,
             nd_v, h01_v, acc0_v, acc1_v,
             st_src, st_dst, st_ew, tmp_v,
             deg_sh, dis_sh):
    cid = lax.axis_index("c")
    sid = lax.axis_index("s")
    wid = cid * _NTILES + sid
    zero16 = jnp.zeros((16,), jnp.float32)
    zi = jnp.zeros((16,), jnp.int32)
    oi = jnp.ones((16,), jnp.int32)

    def _zero(i, _):
        nd_v[pl.ds(i * 16, 16)] = zero16
        acc0_v[pl.ds(i * 16, 16)] = zero16
        acc1_v[pl.ds(i * 16, 16)] = zero16
        return 0
    lax.fori_loop(0, _NP // 16, _zero, 0)

    base_a = sid * _EA

    def _chunk_a(c, _):
        off = base_a + c * _CH
        pltpu.sync_copy(dst_hbm.at[pl.ds(off, _CH)], st_dst)
        pltpu.sync_copy(ew_hbm.at[pl.ds(off, _CH)], st_ew)

        def _grp(j, _):
            d16 = st_dst[pl.ds(j * 16, 16)]
            w16 = st_ew[pl.ds(j * 16, 16)]
            plsc.addupdate_scatter(nd_v, [d16], w16)
            return 0
        lax.fori_loop(0, _GRP, _grp, 0)
        return 0
    lax.fori_loop(0, _EA // _CH, _chunk_a, 0)

    pltpu.sync_copy(nd_v, deg_sh.at[sid])
    plsc.subcore_barrier()

    nbase = sid * _SL
    pltpu.sync_copy(deg_sh.at[0, pl.ds(nbase, _SL)], nd_v.at[pl.ds(nbase, _SL)])
    for t in range(1, _NTILES):
        pltpu.sync_copy(deg_sh.at[t, pl.ds(nbase, _SL)], tmp_v)

        def _acc(g, _):
            off = nbase + g * 16
            nd_v[pl.ds(off, 16)] = (nd_v[pl.ds(off, 16)]
                                    + tmp_v[pl.ds(g * 16, 16)])
            return 0
        lax.fori_loop(0, _SL // 16, _acc, 0)

    def _dis(g, _):
        off = nbase + g * 16
        x = nd_v[pl.ds(off, 16)] + 1.0
        ii = plsc.bitcast(x, jnp.int32)
        ii = jnp.int32(0x5F3759DF) - lax.shift_right_logical(ii, 1)
        y = plsc.bitcast(ii, jnp.float32)
        for _i in range(3):
            y = y * (1.5 - 0.5 * x * y * y)
        nd_v[pl.ds(off, 16)] = y
        return 0
    lax.fori_loop(0, _SL // 16, _dis, 0)

    pltpu.sync_copy(nd_v.at[pl.ds(nbase, _SL)], dis_sh.at[pl.ds(nbase, _SL)])
    plsc.subcore_barrier()
    pltpu.sync_copy(dis_sh, nd_v)

    pltpu.sync_copy(h01_hbm, h01_v)

    @pl.when(cid == 0)
    def _selfloop():
        def _sl_grp(g, _):
            off = nbase + g * 16
            d = nd_v[pl.ds(off, 16)]
            d2 = d * d
            idx = jnp.arange(16, dtype=jnp.int32) + off
            h0 = plsc.load_gather(h01_v, [idx, zi])
            h1 = plsc.load_gather(h01_v, [idx, oi])
            acc0_v[pl.ds(off, 16)] = d2 * h0
            acc1_v[pl.ds(off, 16)] = d2 * h1
            return 0
        lax.fori_loop(0, _SL // 16, _sl_grp, 0)

    base_b = wid * _EB

    def _chunk_b(c, _):
        off = base_b + c * _CH
        pltpu.sync_copy(src_hbm.at[pl.ds(off, _CH)], st_src)
        pltpu.sync_copy(dst_hbm.at[pl.ds(off, _CH)], st_dst)
        pltpu.sync_copy(ew_hbm.at[pl.ds(off, _CH)], st_ew)

        def _grp(j, _):
            s16 = st_src[pl.ds(j * 16, 16)]
            d16 = st_dst[pl.ds(j * 16, 16)]
            w16 = st_ew[pl.ds(j * 16, 16)]
            dsrc = plsc.load_gather(nd_v, [s16])
            ddst = plsc.load_gather(nd_v, [d16])
            nrm = dsrc * w16 * ddst
            h0 = plsc.load_gather(h01_v, [s16, zi])
            h1 = plsc.load_gather(h01_v, [s16, oi])
            plsc.addupdate_scatter(acc0_v, [d16], h0 * nrm)
            plsc.addupdate_scatter(acc1_v, [d16], h1 * nrm)
            return 0
        lax.fori_loop(0, _GRP, _grp, 0)
        return 0
    lax.fori_loop(0, _EB // _CH, _chunk_b, 0)

    pltpu.sync_copy(acc0_v, acc_hbm.at[0, wid])
    pltpu.sync_copy(acc1_v, acc_hbm.at[1, wid])


_sc_agg = functools.partial(
    pl.kernel,
    mesh=plsc.VectorSubcoreMesh(core_axis_name="c", subcore_axis_name="s",
                                num_cores=_NCORES, num_subcores=_NTILES),
    out_type=jax.ShapeDtypeStruct((2, _NW, _NP), jnp.float32),
    compiler_params=pltpu.CompilerParams(needs_layout_passes=False,
                                         use_tc_tiling_on_sc=False),
    scratch_types=[
        pltpu.VMEM((_NP,), jnp.float32),
        pltpu.VMEM((_NP, 2), jnp.float32),
        pltpu.VMEM((_NP,), jnp.float32),
        pltpu.VMEM((_NP,), jnp.float32),
        pltpu.VMEM((_CH,), jnp.int32),
        pltpu.VMEM((_CH,), jnp.int32),
        pltpu.VMEM((_CH,), jnp.float32),
        pltpu.VMEM((_SL,), jnp.float32),
        pltpu.VMEM_SHARED((_NTILES, _NP), jnp.float32),
        pltpu.VMEM_SHARED((_NP,), jnp.float32),
    ],
)(_sc_body)


def kernel(x, edge_list, edge_attr, W1, b1, Wfc, bfc):
    src = edge_list[0].astype(jnp.int32)
    dst = edge_list[1].astype(jnp.int32)
    ew = edge_attr.astype(jnp.float32)
    xp = jnp.pad(x, ((0, _NP - _N), (0, 0)))

    h01 = pl.pallas_call(
        _mm_body,
        out_shape=jax.ShapeDtypeStruct((_NP, 2), jnp.float32),
    )(xp, W1)

    acc = _sc_agg(src, dst, ew, h01)

    wfc2 = jnp.pad(Wfc.reshape(_N, 2).T, ((0, 0), (0, _NP - _N)))
    res = pl.pallas_call(
        _fin_body,
        out_shape=jax.ShapeDtypeStruct((1, 1), jnp.float32),
    )(acc, wfc2, b1.reshape(2, 1), bfc.reshape(1, 1))
    return res[0, 0]
